# Initial kernel scaffold; baseline (speedup 1.0000x reference)
import jax
import jax.numpy as jnp
from jax.experimental import pallas as pl


def kernel(X, edge_index, W, b):
    raise NotImplementedError("write your pallas kernel here")


# speedup vs baseline: 22.9844x; 22.9844x over previous
import functools

import jax
import jax.numpy as jnp
from jax import lax
from jax.experimental import pallas as pl
from jax.experimental.pallas import tpu as pltpu
from jax.experimental.pallas import tpu_sc as plsc

N = 10000
E = 320000
D = 128
NC = 2
NS = 16
NW = NC * NS
CHUNK = 128
CPW = 79
E_PAD = NW * CPW * CHUNK
N_PAD = 10112
ROW_CHUNKS = N_PAD // CHUNK
PAD_ROWS = N_PAD - N
ZJ = -(-ROW_CHUNKS // NS)

_sc_mesh = plsc.VectorSubcoreMesh(
    core_axis_name="c", subcore_axis_name="s", num_cores=NC, num_subcores=NS
)


@functools.partial(
    pl.kernel,
    out_type=(
        jax.ShapeDtypeStruct((N_PAD, 16), jnp.float32),
        jax.ShapeDtypeStruct((N_PAD, 16), jnp.float32),
    ),
    mesh=_sc_mesh,
    scratch_types=[
        pltpu.VMEM((CPW, CHUNK), jnp.int32),
        pltpu.VMEM((CHUNK, 16), jnp.float32),
        pltpu.VMEM((CHUNK, 16), jnp.float32),
        pltpu.VMEM_SHARED((N_PAD, 16), jnp.float32),
    ],
)
def _deg_pass(dst_hbm, consts_hbm, deg0_hbm, deg1_hbm, dst_v, ones_v, zero_v, deg_sh):
    c = lax.axis_index("c")
    s = lax.axis_index("s")
    w = c * NS + s
    pltpu.sync_copy(dst_hbm.at[w], dst_v)
    pltpu.sync_copy(consts_hbm.at[0], ones_v)
    pltpu.sync_copy(consts_hbm.at[1], zero_v)
    for jj in range(ZJ):
        j = jj * NS + s

        @pl.when(j < ROW_CHUNKS)
        def _():
            pltpu.sync_copy(zero_v, deg_sh.at[pl.ds(j * CHUNK, CHUNK)])

    plsc.subcore_barrier()

    @pl.loop(0, CPW)
    def _(j):
        pltpu.sync_copy(ones_v, deg_sh.at[dst_v.at[j]], add=True)

    plsc.subcore_barrier()
    for jj in range(ZJ):
        j = jj * NS + s

        @pl.when(j < ROW_CHUNKS)
        def _():
            sl = pl.ds(j * CHUNK, CHUNK)

            @pl.when(c == 0)
            def _():
                pltpu.sync_copy(deg_sh.at[sl], deg0_hbm.at[sl])

            @pl.when(c == 1)
            def _():
                pltpu.sync_copy(deg_sh.at[sl], deg1_hbm.at[sl])


@functools.partial(
    pl.kernel,
    out_type=(
        jax.ShapeDtypeStruct((N_PAD, D), jnp.float32),
        jax.ShapeDtypeStruct((N_PAD, D), jnp.float32),
    ),
    mesh=_sc_mesh,
    scratch_types=[
        pltpu.VMEM((CPW, CHUNK), jnp.int32),
        pltpu.VMEM((CPW, CHUNK), jnp.int32),
        pltpu.VMEM((CHUNK, D), jnp.float32),
        pltpu.VMEM_SHARED((N_PAD, D), jnp.float32),
    ],
)
def _edge_pass(g_hbm, src_hbm, dst_hbm, zrow_hbm, out0_hbm, out1_hbm,
               src_v, dst_v, rbuf_v, out_sh):
    c = lax.axis_index("c")
    s = lax.axis_index("s")
    w = c * NS + s
    pltpu.sync_copy(src_hbm.at[w], src_v)
    pltpu.sync_copy(dst_hbm.at[w], dst_v)
    pltpu.sync_copy(zrow_hbm, rbuf_v)
    for jj in range(ZJ):
        j = jj * NS + s

        @pl.when(j < ROW_CHUNKS)
        def _():
            pltpu.sync_copy(rbuf_v, out_sh.at[pl.ds(j * CHUNK, CHUNK)])

    plsc.subcore_barrier()

    @pl.loop(0, CPW)
    def _(j):
        pltpu.sync_copy(g_hbm.at[src_v.at[j]], rbuf_v)
        pltpu.sync_copy(rbuf_v, out_sh.at[dst_v.at[j]], add=True)

    plsc.subcore_barrier()
    for jj in range(ZJ):
        j = jj * NS + s

        @pl.when(j < ROW_CHUNKS)
        def _():
            sl = pl.ds(j * CHUNK, CHUNK)

            @pl.when(c == 0)
            def _():
                pltpu.sync_copy(out_sh.at[sl], out0_hbm.at[sl])

            @pl.when(c == 1)
            def _():
                pltpu.sync_copy(out_sh.at[sl], out1_hbm.at[sl])


def _mm_body(x_ref, w_ref, b_ref, d0_ref, d1_ref, g_ref):
    deg = d0_ref[...][:, :1] + d1_ref[...][:, :1] + 1.0
    dinv = lax.rsqrt(deg)
    h = lax.dot_general(
        x_ref[...], w_ref[...], (((1,), (1,)), ((), ())),
        preferred_element_type=jnp.float32,
    )
    g_ref[...] = (h + b_ref[...]) * dinv


def _fin_body(p0_ref, p1_ref, g_ref, d0_ref, d1_ref, o_ref):
    dinv = lax.rsqrt(d0_ref[...][:, :1] + d1_ref[...][:, :1] + 1.0)
    acc = (p0_ref[...] + p1_ref[...] + g_ref[...]) * dinv
    o_ref[...] = jnp.maximum(acc, 0.0)


def kernel(X, edge_index, W, b):
    src = edge_index[0].astype(jnp.int32)
    dst = edge_index[1].astype(jnp.int32)
    pad = (jnp.arange(E_PAD - E, dtype=jnp.int32) % PAD_ROWS) + N
    src_t = jnp.concatenate([src, pad]).reshape(NW, CPW, CHUNK)
    dst_t = jnp.concatenate([dst, pad]).reshape(NW, CPW, CHUNK)
    x_pad = jnp.pad(X, ((0, N_PAD - N), (0, 0)))
    consts = jnp.stack(
        [jnp.ones((CHUNK, 16), jnp.float32), jnp.zeros((CHUNK, 16), jnp.float32)]
    )
    zrow = jnp.zeros((CHUNK, D), jnp.float32)

    deg0, deg1 = _deg_pass(dst_t, consts)

    g = pl.pallas_call(
        _mm_body,
        grid=(ROW_CHUNKS,),
        in_specs=[
            pl.BlockSpec((CHUNK, D), lambda i: (i, 0)),
            pl.BlockSpec((D, D), lambda i: (0, 0)),
            pl.BlockSpec((1, D), lambda i: (0, 0)),
            pl.BlockSpec((CHUNK, 16), lambda i: (i, 0)),
            pl.BlockSpec((CHUNK, 16), lambda i: (i, 0)),
        ],
        out_specs=pl.BlockSpec((CHUNK, D), lambda i: (i, 0)),
        out_shape=jax.ShapeDtypeStruct((N_PAD, D), jnp.float32),
    )(x_pad, W, b.reshape(1, D), deg0, deg1)

    p0, p1 = _edge_pass(g, src_t, dst_t, zrow)

    out_full = pl.pallas_call(
        _fin_body,
        grid=(ROW_CHUNKS,),
        in_specs=[
            pl.BlockSpec((CHUNK, D), lambda i: (i, 0)),
            pl.BlockSpec((CHUNK, D), lambda i: (i, 0)),
            pl.BlockSpec((CHUNK, D), lambda i: (i, 0)),
            pl.BlockSpec((CHUNK, 16), lambda i: (i, 0)),
            pl.BlockSpec((CHUNK, 16), lambda i: (i, 0)),
        ],
        out_specs=pl.BlockSpec((CHUNK, D), lambda i: (i, 0)),
        out_shape=jax.ShapeDtypeStruct((N_PAD, D), jnp.float32),
    )(p0, p1, g, deg0, deg1)

    return out_full[:N]


# speedup vs baseline: 26.7741x; 1.1649x over previous
import functools

import jax
import jax.numpy as jnp
from jax import lax
from jax.experimental import pallas as pl
from jax.experimental.pallas import tpu as pltpu
from jax.experimental.pallas import tpu_sc as plsc

N = 10000
E = 320000
D = 128
NC = 2
NS = 16
NW = NC * NS
CHUNK = 128
CPW = 80
HPW = CPW // 2
E_PAD = NW * CPW * CHUNK
N_PAD = 10112
BRC = N_PAD // CHUNK
PAD_ROWS = N_PAD - N
ZJ = -(-BRC // NS)
DEG_RING = 8

_sc_mesh = plsc.VectorSubcoreMesh(
    core_axis_name="c", subcore_axis_name="s", num_cores=NC, num_subcores=NS
)


@functools.partial(
    pl.kernel,
    out_type=(
        jax.ShapeDtypeStruct((N_PAD, 16), jnp.float32),
        jax.ShapeDtypeStruct((N_PAD, 16), jnp.float32),
    ),
    mesh=_sc_mesh,
    scratch_types=[
        pltpu.VMEM((CPW, CHUNK), jnp.int32),
        pltpu.VMEM((CHUNK, 16), jnp.float32),
        pltpu.VMEM((CHUNK, 16), jnp.float32),
        pltpu.VMEM_SHARED((N_PAD, 16), jnp.float32),
        pltpu.SemaphoreType.DMA,
    ],
)
def _deg_pass(dst_hbm, consts_hbm, deg0_hbm, deg1_hbm,
              dst_v, ones_v, zero_v, deg_sh, dsem):
    c = lax.axis_index("c")
    s = lax.axis_index("s")
    w = c * NS + s
    pltpu.sync_copy(dst_hbm.at[w], dst_v)
    pltpu.sync_copy(consts_hbm.at[0], ones_v)
    pltpu.sync_copy(consts_hbm.at[1], zero_v)
    for jj in range(ZJ):
        j = jj * NS + s

        @pl.when(j < BRC)
        def _():
            pltpu.sync_copy(zero_v, deg_sh.at[pl.ds(j * CHUNK, CHUNK)])

    plsc.subcore_barrier()

    for j in range(DEG_RING):
        pltpu.async_copy(ones_v, deg_sh.at[dst_v.at[j]], dsem, add=True)

    @pl.loop(0, CPW)
    def _(j):
        pltpu.make_async_copy(ones_v, deg_sh.at[dst_v.at[j]], dsem).wait()

        @pl.when(j + DEG_RING < CPW)
        def _():
            pltpu.async_copy(
                ones_v, deg_sh.at[dst_v.at[j + DEG_RING]], dsem, add=True
            )

    plsc.subcore_barrier()
    for jj in range(ZJ):
        j = jj * NS + s

        @pl.when(j < BRC)
        def _():
            sl = pl.ds(j * CHUNK, CHUNK)

            @pl.when(c == 0)
            def _():
                pltpu.sync_copy(deg_sh.at[sl], deg0_hbm.at[sl])

            @pl.when(c == 1)
            def _():
                pltpu.sync_copy(deg_sh.at[sl], deg1_hbm.at[sl])


@functools.partial(
    pl.kernel,
    out_type=(
        jax.ShapeDtypeStruct((N_PAD, D), jnp.float32),
        jax.ShapeDtypeStruct((N_PAD, D), jnp.float32),
    ),
    mesh=_sc_mesh,
    scratch_types=[
        pltpu.VMEM((HPW, CHUNK), jnp.int32),
        pltpu.VMEM((HPW, CHUNK), jnp.int32),
        pltpu.VMEM((2, CHUNK, D), jnp.float32),
        pltpu.VMEM_SHARED((N_PAD, D), jnp.float32),
        pltpu.SemaphoreType.DMA,
        pltpu.SemaphoreType.DMA,
    ],
)
def _edge_pass(g_hbm, src_hbm, dst_hbm, zrow_hbm, out0_hbm, out1_hbm,
               src_v, dst_v, rbuf_v, out_sh, gsem0, gsem1):
    c = lax.axis_index("c")
    s = lax.axis_index("s")
    w = c * NS + s
    pltpu.sync_copy(zrow_hbm, rbuf_v.at[0])
    for jj in range(ZJ):
        j = jj * NS + s

        @pl.when(j < BRC)
        def _():
            pltpu.sync_copy(rbuf_v.at[0], out_sh.at[pl.ds(j * CHUNK, CHUNK)])

    plsc.subcore_barrier()

    for p in range(2):
        pltpu.sync_copy(src_hbm.at[w * 2 + p], src_v)
        pltpu.sync_copy(dst_hbm.at[w * 2 + p], dst_v)
        pltpu.async_copy(g_hbm.at[src_v.at[0]], rbuf_v.at[0], gsem0)

        @pl.loop(0, HPW, step=2)
        def _(j):
            for b in range(2):
                jj = j + b
                sem_b = gsem0 if b == 0 else gsem1
                sem_o = gsem1 if b == 0 else gsem0
                pltpu.make_async_copy(
                    g_hbm.at[src_v.at[jj]], rbuf_v.at[b], sem_b
                ).wait()

                @pl.when(jj + 1 < HPW)
                def _():
                    pltpu.async_copy(
                        g_hbm.at[src_v.at[jj + 1]], rbuf_v.at[1 - b], sem_o
                    )

                pltpu.sync_copy(
                    rbuf_v.at[b], out_sh.at[dst_v.at[jj]], add=True
                )

    plsc.subcore_barrier()
    for jj in range(ZJ):
        j = jj * NS + s

        @pl.when(j < BRC)
        def _():
            sl = pl.ds(j * CHUNK, CHUNK)

            @pl.when(c == 0)
            def _():
                pltpu.sync_copy(out_sh.at[sl], out0_hbm.at[sl])

            @pl.when(c == 1)
            def _():
                pltpu.sync_copy(out_sh.at[sl], out1_hbm.at[sl])


def _mm_body(x_ref, w_ref, b_ref, d0_ref, d1_ref, g_ref):
    deg = d0_ref[...][:, :1] + d1_ref[...][:, :1] + 1.0
    dinv = lax.rsqrt(deg)
    h = lax.dot_general(
        x_ref[...], w_ref[...], (((1,), (1,)), ((), ())),
        preferred_element_type=jnp.float32,
    )
    g_ref[...] = (h + b_ref[...]) * dinv


def _fin_body(p0_ref, p1_ref, g_ref, d0_ref, d1_ref, o_ref):
    dinv = lax.rsqrt(d0_ref[...][:, :1] + d1_ref[...][:, :1] + 1.0)
    acc = (p0_ref[...] + p1_ref[...] + g_ref[...]) * dinv
    o_ref[...] = jnp.maximum(acc, 0.0)


def kernel(X, edge_index, W, b):
    src = edge_index[0].astype(jnp.int32)
    dst = edge_index[1].astype(jnp.int32)
    pad = (jnp.arange(E_PAD - E, dtype=jnp.int32) % PAD_ROWS) + N
    src_t = jnp.concatenate([src, pad]).reshape(NW, CPW, CHUNK)
    dst_t = jnp.concatenate([dst, pad]).reshape(NW, CPW, CHUNK)
    src_t2 = src_t.reshape(NW * 2, HPW, CHUNK)
    dst_t2 = dst_t.reshape(NW * 2, HPW, CHUNK)
    x_pad = jnp.pad(X, ((0, N_PAD - N), (0, 0)))
    consts = jnp.stack(
        [jnp.ones((CHUNK, 16), jnp.float32), jnp.zeros((CHUNK, 16), jnp.float32)]
    )
    zrow = jnp.zeros((CHUNK, D), jnp.float32)

    deg0, deg1 = _deg_pass(dst_t, consts)

    g = pl.pallas_call(
        _mm_body,
        grid=(BRC,),
        in_specs=[
            pl.BlockSpec((CHUNK, D), lambda i: (i, 0)),
            pl.BlockSpec((D, D), lambda i: (0, 0)),
            pl.BlockSpec((1, D), lambda i: (0, 0)),
            pl.BlockSpec((CHUNK, 16), lambda i: (i, 0)),
            pl.BlockSpec((CHUNK, 16), lambda i: (i, 0)),
        ],
        out_specs=pl.BlockSpec((CHUNK, D), lambda i: (i, 0)),
        out_shape=jax.ShapeDtypeStruct((N_PAD, D), jnp.float32),
    )(x_pad, W, b.reshape(1, D), deg0, deg1)

    p0, p1 = _edge_pass(g, src_t2, dst_t2, zrow)

    out_full = pl.pallas_call(
        _fin_body,
        grid=(BRC,),
        in_specs=[
            pl.BlockSpec((CHUNK, D), lambda i: (i, 0)),
            pl.BlockSpec((CHUNK, D), lambda i: (i, 0)),
            pl.BlockSpec((CHUNK, D), lambda i: (i, 0)),
            pl.BlockSpec((CHUNK, 16), lambda i: (i, 0)),
            pl.BlockSpec((CHUNK, 16), lambda i: (i, 0)),
        ],
        out_specs=pl.BlockSpec((CHUNK, D), lambda i: (i, 0)),
        out_shape=jax.ShapeDtypeStruct((N_PAD, D), jnp.float32),
    )(p0, p1, g, deg0, deg1)

    return out_full[:N]


# speedup vs baseline: 36.7170x; 1.3714x over previous
import functools

import jax
import jax.numpy as jnp
from jax import lax
from jax.experimental import pallas as pl
from jax.experimental.pallas import tpu as pltpu
from jax.experimental.pallas import tpu_sc as plsc

N = 10000
E = 320000
D = 128
NC = 2
NS = 16
NW = NC * NS
CHUNK = 128
CPW = 80
HPW = CPW // 2
E_PAD = NW * CPW * CHUNK
N_PAD = 10112
BRC = N_PAD // CHUNK
PAD_ROWS = N_PAD - N
ZJ = -(-BRC // NS)
DEG_RING = 8
TC_ROWS = 1264
TC_GRID = N_PAD // TC_ROWS

_sc_mesh = plsc.VectorSubcoreMesh(
    core_axis_name="c", subcore_axis_name="s", num_cores=NC, num_subcores=NS
)


@functools.partial(
    pl.kernel,
    out_type=(
        jax.ShapeDtypeStruct((N_PAD, 16), jnp.float32),
        jax.ShapeDtypeStruct((N_PAD, 16), jnp.float32),
    ),
    mesh=_sc_mesh,
    scratch_types=[
        pltpu.VMEM((CPW, CHUNK), jnp.int32),
        pltpu.VMEM((CHUNK, 16), jnp.float32),
        pltpu.VMEM((CHUNK, 16), jnp.float32),
        pltpu.VMEM_SHARED((N_PAD, 16), jnp.float32),
        pltpu.SemaphoreType.DMA,
    ],
)
def _deg_pass(dst_hbm, consts_hbm, deg0_hbm, deg1_hbm,
              dst_v, ones_v, zero_v, deg_sh, dsem):
    c = lax.axis_index("c")
    s = lax.axis_index("s")
    w = c * NS + s
    pltpu.sync_copy(dst_hbm.at[w], dst_v)
    pltpu.sync_copy(consts_hbm.at[0], ones_v)
    pltpu.sync_copy(consts_hbm.at[1], zero_v)
    for jj in range(ZJ):
        j = jj * NS + s

        @pl.when(j < BRC)
        def _():
            pltpu.sync_copy(zero_v, deg_sh.at[pl.ds(j * CHUNK, CHUNK)])

    plsc.subcore_barrier()

    for j in range(DEG_RING):
        pltpu.async_copy(ones_v, deg_sh.at[dst_v.at[j]], dsem, add=True)

    @pl.loop(0, CPW)
    def _(j):
        pltpu.make_async_copy(ones_v, deg_sh.at[dst_v.at[j]], dsem).wait()

        @pl.when(j + DEG_RING < CPW)
        def _():
            pltpu.async_copy(
                ones_v, deg_sh.at[dst_v.at[j + DEG_RING]], dsem, add=True
            )

    plsc.subcore_barrier()
    for jj in range(ZJ):
        j = jj * NS + s

        @pl.when(j < BRC)
        def _():
            sl = pl.ds(j * CHUNK, CHUNK)

            @pl.when(c == 0)
            def _():
                pltpu.sync_copy(deg_sh.at[sl], deg0_hbm.at[sl])

            @pl.when(c == 1)
            def _():
                pltpu.sync_copy(deg_sh.at[sl], deg1_hbm.at[sl])


@functools.partial(
    pl.kernel,
    out_type=(
        jax.ShapeDtypeStruct((N_PAD, D), jnp.float32),
        jax.ShapeDtypeStruct((N_PAD, D), jnp.float32),
    ),
    mesh=_sc_mesh,
    scratch_types=[
        pltpu.VMEM((HPW, CHUNK), jnp.int32),
        pltpu.VMEM((HPW, CHUNK), jnp.int32),
        pltpu.VMEM((2, CHUNK, D), jnp.float32),
        pltpu.VMEM_SHARED((N_PAD, D), jnp.float32),
        pltpu.SemaphoreType.DMA,
        pltpu.SemaphoreType.DMA,
    ],
)
def _edge_pass(g_hbm, src_hbm, dst_hbm, zrow_hbm, out0_hbm, out1_hbm,
               src_v, dst_v, rbuf_v, out_sh, gsem0, gsem1):
    c = lax.axis_index("c")
    s = lax.axis_index("s")
    w = c * NS + s
    pltpu.sync_copy(zrow_hbm, rbuf_v.at[0])
    for jj in range(ZJ):
        j = jj * NS + s

        @pl.when(j < BRC)
        def _():
            pltpu.sync_copy(rbuf_v.at[0], out_sh.at[pl.ds(j * CHUNK, CHUNK)])

    plsc.subcore_barrier()

    for p in range(2):
        pltpu.sync_copy(src_hbm.at[w * 2 + p], src_v)
        pltpu.sync_copy(dst_hbm.at[w * 2 + p], dst_v)
        pltpu.async_copy(g_hbm.at[src_v.at[0]], rbuf_v.at[0], gsem0)

        @pl.loop(0, HPW, step=2)
        def _(j):
            for b in range(2):
                jj = j + b
                sem_b = gsem0 if b == 0 else gsem1
                sem_o = gsem1 if b == 0 else gsem0
                pltpu.make_async_copy(
                    g_hbm.at[src_v.at[jj]], rbuf_v.at[b], sem_b
                ).wait()

                @pl.when(jj + 1 < HPW)
                def _():
                    pltpu.async_copy(
                        g_hbm.at[src_v.at[jj + 1]], rbuf_v.at[1 - b], sem_o
                    )

                pltpu.sync_copy(
                    rbuf_v.at[b], out_sh.at[dst_v.at[jj]], add=True
                )

    plsc.subcore_barrier()
    for jj in range(ZJ):
        j = jj * NS + s

        @pl.when(j < BRC)
        def _():
            sl = pl.ds(j * CHUNK, CHUNK)

            @pl.when(c == 0)
            def _():
                pltpu.sync_copy(out_sh.at[sl], out0_hbm.at[sl])

            @pl.when(c == 1)
            def _():
                pltpu.sync_copy(out_sh.at[sl], out1_hbm.at[sl])


def _mm_body(x_ref, w_ref, b_ref, d0_ref, d1_ref, g_ref):
    deg = d0_ref[...][:, :1] + d1_ref[...][:, :1] + 1.0
    dinv = lax.rsqrt(deg)
    h = lax.dot_general(
        x_ref[...], w_ref[...], (((1,), (1,)), ((), ())),
        preferred_element_type=jnp.float32,
    )
    g_ref[...] = (h + b_ref[...]) * dinv


def _fin_body(p0_ref, p1_ref, g_ref, d0_ref, d1_ref, o_ref):
    dinv = lax.rsqrt(d0_ref[...][:, :1] + d1_ref[...][:, :1] + 1.0)
    acc = (p0_ref[...] + p1_ref[...] + g_ref[...]) * dinv
    o_ref[...] = jnp.maximum(acc, 0.0)


def kernel(X, edge_index, W, b):
    pad = (jnp.arange(E_PAD - E, dtype=jnp.int32) % PAD_ROWS) + N
    pad2 = jnp.broadcast_to(pad, (2, E_PAD - E))
    ei = jnp.concatenate([edge_index.astype(jnp.int32), pad2], axis=1)
    src_t = ei[0].reshape(NW, CPW, CHUNK)
    dst_t = ei[1].reshape(NW, CPW, CHUNK)
    src_t2 = src_t.reshape(NW * 2, HPW, CHUNK)
    dst_t2 = dst_t.reshape(NW * 2, HPW, CHUNK)
    x_pad = jnp.pad(X, ((0, N_PAD - N), (0, 0)))
    consts = jnp.stack(
        [jnp.ones((CHUNK, 16), jnp.float32), jnp.zeros((CHUNK, 16), jnp.float32)]
    )
    zrow = jnp.zeros((CHUNK, D), jnp.float32)

    deg0, deg1 = _deg_pass(dst_t, consts)

    g = pl.pallas_call(
        _mm_body,
        grid=(TC_GRID,),
        in_specs=[
            pl.BlockSpec((TC_ROWS, D), lambda i: (i, 0)),
            pl.BlockSpec((D, D), lambda i: (0, 0)),
            pl.BlockSpec((1, D), lambda i: (0, 0)),
            pl.BlockSpec((TC_ROWS, 16), lambda i: (i, 0)),
            pl.BlockSpec((TC_ROWS, 16), lambda i: (i, 0)),
        ],
        out_specs=pl.BlockSpec((TC_ROWS, D), lambda i: (i, 0)),
        out_shape=jax.ShapeDtypeStruct((N_PAD, D), jnp.float32),
    )(x_pad, W, b.reshape(1, D), deg0, deg1)

    p0, p1 = _edge_pass(g, src_t2, dst_t2, zrow)

    out_full = pl.pallas_call(
        _fin_body,
        grid=(TC_GRID,),
        in_specs=[
            pl.BlockSpec((TC_ROWS, D), lambda i: (i, 0)),
            pl.BlockSpec((TC_ROWS, D), lambda i: (i, 0)),
            pl.BlockSpec((TC_ROWS, D), lambda i: (i, 0)),
            pl.BlockSpec((TC_ROWS, 16), lambda i: (i, 0)),
            pl.BlockSpec((TC_ROWS, 16), lambda i: (i, 0)),
        ],
        out_specs=pl.BlockSpec((TC_ROWS, D), lambda i: (i, 0)),
        out_shape=jax.ShapeDtypeStruct((N_PAD, D), jnp.float32),
    )(p0, p1, g, deg0, deg1)

    return out_full[:N]
